# baseline (speedup 1.0000x reference)
import jax
import jax.numpy as jnp
from jax.experimental import pallas as pl
from jax.experimental.pallas import tpu as pltpu

NB = 4
TOP_K = 2


def _moe_body(x_ref, wg_ref, w1_ref, w3_ref, w2_ref, out_ref, route_ref,
              a_ref):
    e = pl.program_id(0)
    p = pl.program_id(1)
    nb = pl.program_id(2)
    f_blk = a_ref.shape[1] // NB
    d_blk = out_ref.shape[1] // NB

    @pl.when(jnp.logical_and(e == 0, jnp.logical_and(p == 0, nb == 0)))
    def _init():
        xv = x_ref[...]
        logits = jax.lax.dot_general(
            xv, wg_ref[...], (((1,), (0,)), ((), ())),
            preferred_element_type=jnp.float32)
        mx = jnp.max(logits, axis=-1, keepdims=True)
        pr = jnp.exp(logits - mx)
        pr = pr / jnp.sum(pr, axis=-1, keepdims=True)
        ecols = jax.lax.broadcasted_iota(jnp.int32, pr.shape, 1)
        m1 = jnp.max(pr, axis=-1, keepdims=True)
        i1 = jnp.argmax(pr, axis=-1)[:, None]
        masked = jnp.where(ecols == i1, -jnp.inf, pr)
        m2 = jnp.max(masked, axis=-1, keepdims=True)
        i2 = jnp.argmax(masked, axis=-1)[:, None]
        s = m1 + m2
        route_ref[...] = jnp.where(
            ecols == i1, m1 / s, jnp.where(ecols == i2, m2 / s, 0.0))
        out_ref[...] = jnp.zeros_like(out_ref)

    @pl.when(p == 0)
    def _up_proj():
        xv = x_ref[...]
        g = jax.lax.dot_general(
            xv, w1_ref[0], (((1,), (1,)), ((), ())),
            preferred_element_type=jnp.float32)
        u = jax.lax.dot_general(
            xv, w3_ref[0], (((1,), (1,)), ((), ())),
            preferred_element_type=jnp.float32)
        ecols = jax.lax.broadcasted_iota(jnp.int32, route_ref.shape, 1)
        rw = jnp.sum(jnp.where(ecols == e, route_ref[...], 0.0), axis=1,
                     keepdims=True)
        a_ref[:, pl.ds(nb * f_blk, f_blk)] = (g * jax.lax.logistic(g)) * u * rw

    @pl.when(p == 1)
    def _down_proj():
        y = jax.lax.dot_general(
            a_ref[...], w2_ref[0], (((1,), (1,)), ((), ())),
            preferred_element_type=jnp.float32)
        out_ref[:, pl.ds(nb * d_blk, d_blk)] += y


@jax.jit
def kernel(x, Wg, w1, w3, w2):
    m, d = x.shape
    e_num = Wg.shape[1]
    f = w1.shape[1]
    f_blk = f // NB
    d_blk = d // NB
    return pl.pallas_call(
        _moe_body,
        grid=(e_num, 2, NB),
        in_specs=[
            pl.BlockSpec((m, d), lambda e, p, nb: (0, 0)),
            pl.BlockSpec((d, e_num), lambda e, p, nb: (0, 0)),
            pl.BlockSpec((1, f_blk, d),
                         lambda e, p, nb: (e, jnp.where(p == 0, nb, NB - 1), 0)),
            pl.BlockSpec((1, f_blk, d),
                         lambda e, p, nb: (e, jnp.where(p == 0, nb, NB - 1), 0)),
            pl.BlockSpec((1, d_blk, f),
                         lambda e, p, nb: (e, jnp.where(p == 1, nb, 0), 0)),
        ],
        out_specs=pl.BlockSpec((m, d), lambda e, p, nb: (0, 0)),
        out_shape=jax.ShapeDtypeStruct((m, d), x.dtype),
        scratch_shapes=[
            pltpu.VMEM((m, e_num), jnp.float32),
            pltpu.VMEM((m, f), jnp.float32),
        ],
    )(x, Wg, w1, w3, w2)


# speedup vs baseline: 1.2219x; 1.2219x over previous
import jax
import jax.numpy as jnp
from jax.experimental import pallas as pl
from jax.experimental.pallas import tpu as pltpu

NB = 4
TOP_K = 2


def _moe_body(x_ref, wg_ref, w1_ref, w3_ref, w2_ref, out_ref, route_ref,
              a_ref):
    e = pl.program_id(0)
    nb = pl.program_id(1)
    n_e = route_ref.shape[1]
    f = a_ref.shape[2]
    f_blk = f // NB
    d_blk = out_ref.shape[1] // NB

    @pl.when(jnp.logical_and(e == 0, nb == 0))
    def _init():
        xv = x_ref[...]
        logits = jax.lax.dot_general(
            xv, wg_ref[...], (((1,), (0,)), ((), ())),
            preferred_element_type=jnp.float32)
        mx = jnp.max(logits, axis=-1, keepdims=True)
        pr = jnp.exp(logits - mx)
        pr = pr / jnp.sum(pr, axis=-1, keepdims=True)
        ecols = jax.lax.broadcasted_iota(jnp.int32, pr.shape, 1)
        m1 = jnp.max(pr, axis=-1, keepdims=True)
        i1 = jnp.argmax(pr, axis=-1)[:, None]
        masked = jnp.where(ecols == i1, -jnp.inf, pr)
        m2 = jnp.max(masked, axis=-1, keepdims=True)
        i2 = jnp.argmax(masked, axis=-1)[:, None]
        s = m1 + m2
        route_ref[...] = jnp.where(
            ecols == i1, m1 / s, jnp.where(ecols == i2, m2 / s, 0.0))
        out_ref[...] = jnp.zeros_like(out_ref)

    @pl.when(e < n_e)
    def _up_proj():
        xv = x_ref[...]
        g = jax.lax.dot_general(
            xv, w1_ref[0], (((1,), (1,)), ((), ())),
            preferred_element_type=jnp.float32)
        u = jax.lax.dot_general(
            xv, w3_ref[0], (((1,), (1,)), ((), ())),
            preferred_element_type=jnp.float32)
        ecols = jax.lax.broadcasted_iota(jnp.int32, route_ref.shape, 1)
        rw = jnp.sum(jnp.where(ecols == e, route_ref[...], 0.0), axis=1,
                     keepdims=True)
        a_ref[e % 2, :, pl.ds(nb * f_blk, f_blk)] = (
            (g * jax.lax.logistic(g)) * u * rw)

    @pl.when(e > 0)
    def _down_proj():
        a_prev = a_ref[(e - 1) % 2]
        y = jax.lax.dot_general(
            a_prev, w2_ref[0], (((1,), (1,)), ((), ())),
            preferred_element_type=jnp.float32)
        out_ref[:, pl.ds(nb * d_blk, d_blk)] += y


@jax.jit
def kernel(x, Wg, w1, w3, w2):
    m, d = x.shape
    e_num = Wg.shape[1]
    f = w1.shape[1]
    f_blk = f // NB
    d_blk = d // NB
    return pl.pallas_call(
        _moe_body,
        grid=(e_num + 1, NB),
        in_specs=[
            pl.BlockSpec((m, d), lambda e, nb: (0, 0)),
            pl.BlockSpec((d, e_num), lambda e, nb: (0, 0)),
            pl.BlockSpec(
                (1, f_blk, d),
                lambda e, nb: (jnp.minimum(e, e_num - 1),
                               jnp.where(e < e_num, nb, NB - 1), 0)),
            pl.BlockSpec(
                (1, f_blk, d),
                lambda e, nb: (jnp.minimum(e, e_num - 1),
                               jnp.where(e < e_num, nb, NB - 1), 0)),
            pl.BlockSpec(
                (1, d_blk, f),
                lambda e, nb: (jnp.maximum(e - 1, 0),
                               jnp.where(e == 0, 0, nb), 0)),
        ],
        out_specs=pl.BlockSpec((m, d), lambda e, nb: (0, 0)),
        out_shape=jax.ShapeDtypeStruct((m, d), x.dtype),
        scratch_shapes=[
            pltpu.VMEM((m, e_num), jnp.float32),
            pltpu.VMEM((2, m, f), jnp.float32),
        ],
    )(x, Wg, w1, w3, w2)
